# Initial kernel scaffold; baseline (speedup 1.0000x reference)
import jax
import jax.numpy as jnp
from jax.experimental import pallas as pl


def kernel(x, edge_index, edge_type, indice_pairs, W1, W1_root, W2, W2_root):
    raise NotImplementedError("write your pallas kernel here")


# speedup vs baseline: 27.9056x; 27.9056x over previous
import functools

import jax
import jax.numpy as jnp
from jax import lax
from jax.experimental import pallas as pl
from jax.experimental.pallas import tpu as pltpu
from jax.experimental.pallas import tpu_sc as plsc

_NC = 2
_NS = 16
_NW = _NC * _NS
_L = 16


def _mesh():
    return plsc.VectorSubcoreMesh(core_axis_name="c", subcore_axis_name="s")


def _edge_prep(npad, erows, deg_size):
    out_type = (
        jax.ShapeDtypeStruct((erows, 128), jnp.int32),
        jax.ShapeDtypeStruct((erows, 128), jnp.int32),
        jax.ShapeDtypeStruct((erows, 128), jnp.float32),
    )
    zslice = deg_size // _NS

    @functools.partial(
        pl.kernel,
        out_type=out_type,
        mesh=_mesh(),
        scratch_types=[
            pltpu.VMEM_SHARED((deg_size,), jnp.float32),
            pltpu.VMEM((zslice,), jnp.float32),
            pltpu.VMEM((128,), jnp.int32),
            pltpu.VMEM((128,), jnp.int32),
            pltpu.VMEM((128,), jnp.int32),
            pltpu.VMEM((128,), jnp.int32),
            pltpu.VMEM((128,), jnp.int32),
            pltpu.VMEM((128,), jnp.float32),
            pltpu.VMEM((128,), jnp.float32),
            pltpu.VMEM((128,), jnp.float32),
        ],
    )
    def prep(type2d, dst2d, src2d, fsrc_out, dst_out, norm_out,
             deg_sh, zbuf, t_v, d_v, s_v, fdst_v, fsrc_v, ones_v, degv,
             norm_v):
        c = lax.axis_index("c")
        s = lax.axis_index("s")
        wid = s * _NC + c

        def zz(i, carry):
            zbuf[pl.ds(i * _L, _L)] = jnp.zeros((_L,), jnp.float32)
            return carry
        lax.fori_loop(0, zslice // _L, zz, None)

        def oo(i, carry):
            ones_v[pl.ds(i * _L, _L)] = jnp.ones((_L,), jnp.float32)
            return carry
        lax.fori_loop(0, 128 // _L, oo, None)

        pltpu.sync_copy(zbuf, deg_sh.at[pl.ds(s * zslice, zslice)])
        plsc.subcore_barrier()

        nrows_deg = (erows - s + _NS - 1) // _NS

        def deg_body(i, carry):
            row = s + i * _NS
            pltpu.sync_copy(type2d.at[row], t_v)
            pltpu.sync_copy(dst2d.at[row], d_v)
            for j in range(128 // _L):
                sl = pl.ds(j * _L, _L)
                fdst_v[sl] = t_v[sl] * npad + d_v[sl]
            pltpu.sync_copy(ones_v, deg_sh.at[fdst_v], add=True)
            return carry
        lax.fori_loop(0, nrows_deg, deg_body, None)
        plsc.subcore_barrier()

        nrows_w = (erows - wid + _NW - 1) // _NW

        def norm_body(i, carry):
            row = wid + i * _NW
            pltpu.sync_copy(type2d.at[row], t_v)
            pltpu.sync_copy(dst2d.at[row], d_v)
            pltpu.sync_copy(src2d.at[row], s_v)
            for j in range(128 // _L):
                sl = pl.ds(j * _L, _L)
                t = t_v[sl]
                fdst_v[sl] = t * npad + d_v[sl]
                fsrc_v[sl] = t * npad + s_v[sl]
            pltpu.sync_copy(deg_sh.at[fdst_v], degv)
            for j in range(128 // _L):
                sl = pl.ds(j * _L, _L)
                norm_v[sl] = 1.0 / jnp.maximum(degv[sl], 1.0)
            pltpu.sync_copy(fsrc_v, fsrc_out.at[row])
            pltpu.sync_copy(d_v, dst_out.at[row])
            pltpu.sync_copy(norm_v, norm_out.at[row])
            return carry
        lax.fori_loop(0, nrows_w, norm_body, None)

    return prep


def _aggregate(npad, erows_pad, dw):
    rows_per_tile = erows_pad // _NW
    acc_rows = npad // _NS
    nch = dw // _L

    @functools.partial(
        pl.kernel,
        out_type=jax.ShapeDtypeStruct((_NC, npad, dw), jnp.float32),
        mesh=_mesh(),
        scratch_types=[
            pltpu.VMEM_SHARED((npad, dw), jnp.float32),
            pltpu.VMEM((128, dw), jnp.float32),
            pltpu.VMEM((128,), jnp.int32),
            pltpu.VMEM((128,), jnp.int32),
            pltpu.VMEM((128,), jnp.float32),
            pltpu.SemaphoreType.DMA,
        ],
    )
    def agg(h_flat, fsrc_p, dst_p, norm_p, out, acc_sh, rows_v, fsrc_v,
            dst_v, norm_v, sem):
        c = lax.axis_index("c")
        s = lax.axis_index("s")
        wid = s * _NC + c

        def zrow(k, carry):
            for j in range(nch):
                rows_v[k, pl.ds(j * _L, _L)] = jnp.zeros((_L,), jnp.float32)
            return carry
        lax.fori_loop(0, 128, zrow, None)

        def zacc(i, carry):
            pltpu.sync_copy(rows_v,
                            acc_sh.at[pl.ds(s * acc_rows + i * 128, 128)])
            return carry
        lax.fori_loop(0, acc_rows // 128, zacc, None)
        plsc.subcore_barrier()

        def body(i, carry):
            row = wid * rows_per_tile + i
            pltpu.sync_copy(fsrc_p.at[row], fsrc_v)
            pltpu.sync_copy(dst_p.at[row], dst_v)
            pltpu.sync_copy(norm_p.at[row], norm_v)
            pltpu.async_copy(h_flat.at[fsrc_v], rows_v, sem).wait()

            def scale(g, carry2):
                nv16 = norm_v[pl.ds(g * _L, _L)]
                for k16 in range(_L):
                    k = g * _L + k16
                    nvs = jnp.full((_L,), nv16[k16], jnp.float32)
                    for j in range(nch):
                        sl = pl.ds(j * _L, _L)
                        rows_v[k, sl] = rows_v[k, sl] * nvs
                return carry2
            lax.fori_loop(0, 128 // _L, scale, None)
            pltpu.sync_copy(rows_v, acc_sh.at[dst_v], add=True)
            return carry
        lax.fori_loop(0, rows_per_tile, body, None)
        plsc.subcore_barrier()

        def drain(i, carry):
            sl = pl.ds(s * acc_rows + i * 128, 128)
            pltpu.sync_copy(acc_sh.at[sl], out.at[c, sl])
            return carry
        lax.fori_loop(0, acc_rows // 128, drain, None)

    return agg


def _pair_gather(nidx_rows, dw):
    rpt = nidx_rows // _NW

    @functools.partial(
        pl.kernel,
        out_type=jax.ShapeDtypeStruct((nidx_rows * 128, dw), jnp.float32),
        mesh=_mesh(),
        scratch_types=[
            pltpu.VMEM((128,), jnp.int32),
            pltpu.VMEM((128, dw), jnp.float32),
            pltpu.SemaphoreType.DMA,
        ],
    )
    def gk(z_hbm, idx2d, out, idx_v, rows_v, sem):
        c = lax.axis_index("c")
        s = lax.axis_index("s")
        wid = s * _NC + c

        def body(i, carry):
            row = wid * rpt + i
            pltpu.sync_copy(idx2d.at[row], idx_v)
            pltpu.async_copy(z_hbm.at[idx_v], rows_v, sem).wait()
            pltpu.sync_copy(rows_v, out.at[pl.ds(row * 128, 128)])
            return carry
        lax.fori_loop(0, rpt, body, None)

    return gk


def _relmm(nrel, npad, din, dh):
    def mmk(x_ref, w_ref, o_ref):
        for r in range(nrel):
            o_ref[r] = jnp.dot(x_ref[...], w_ref[r],
                               preferred_element_type=jnp.float32)

    return pl.pallas_call(
        mmk,
        grid=(npad // 128,),
        in_specs=[
            pl.BlockSpec((128, din), lambda n: (n, 0)),
            pl.BlockSpec((nrel, din, dh), lambda n: (0, 0, 0)),
        ],
        out_specs=pl.BlockSpec((nrel, 128, dh), lambda n: (0, n, 0)),
        out_shape=jax.ShapeDtypeStruct((nrel, npad, dh), jnp.float32),
    )


def _root_fuse(npad, din, dh, relu):
    def k(x_ref, w_ref, a_ref, b_ref, o_ref):
        acc = jnp.dot(x_ref[...], w_ref[...],
                      preferred_element_type=jnp.float32)
        acc = acc + a_ref[...] + b_ref[...]
        o_ref[...] = jnp.maximum(acc, 0.0) if relu else acc

    return pl.pallas_call(
        k,
        grid=(npad // 128,),
        in_specs=[
            pl.BlockSpec((128, din), lambda n: (n, 0)),
            pl.BlockSpec((din, dh), lambda n: (0, 0)),
            pl.BlockSpec((128, dh), lambda n: (n, 0)),
            pl.BlockSpec((128, dh), lambda n: (n, 0)),
        ],
        out_specs=pl.BlockSpec((128, dh), lambda n: (n, 0)),
        out_shape=jax.ShapeDtypeStruct((npad, dh), jnp.float32),
    )


def kernel(x, edge_index, edge_type, indice_pairs, W1, W1_root, W2, W2_root):
    n_nodes, din = x.shape
    n_edges = edge_type.shape[0]
    nrel = W1.shape[0]
    dh = W1.shape[2]
    dout = W2.shape[2]
    npairs = indice_pairs.shape[0]

    assert n_edges % 128 == 0 and (2 * npairs) % (128 * _NW) == 0
    npad = -(-n_nodes // 2048) * 2048
    erows = n_edges // 128
    erows_pad = -(-erows // _NW) * _NW
    dwout = -(-dout // 128) * 128
    deg_size = nrel * npad

    x_p = jnp.pad(x, ((0, npad - n_nodes), (0, 0)))
    type2d = edge_type.reshape(erows, 128)
    src2d = edge_index[0].reshape(erows, 128)
    dst2d = edge_index[1].reshape(erows, 128)
    w2_p = jnp.pad(W2, ((0, 0), (0, 0), (0, dwout - dout)))
    w2r_p = jnp.pad(W2_root, ((0, 0), (0, dwout - dout)))

    fsrc_p, dst_p, norm_p = _edge_prep(npad, erows, deg_size)(
        type2d, dst2d, src2d)

    pad_rows = erows_pad - erows
    if pad_rows:
        iota_row = jnp.tile(jnp.arange(128, dtype=jnp.int32), (pad_rows, 1))
        fsrc_p = jnp.concatenate([fsrc_p, iota_row], axis=0)
        dst_p = jnp.concatenate([dst_p, iota_row], axis=0)
        norm_p = jnp.concatenate(
            [norm_p, jnp.zeros((pad_rows, 128), jnp.float32)], axis=0)

    h1 = _relmm(nrel, npad, din, dh)(x_p, W1)
    acc1 = _aggregate(npad, erows_pad, dh)(
        h1.reshape(nrel * npad, dh), fsrc_p, dst_p, norm_p)
    h = _root_fuse(npad, din, dh, True)(x_p, W1_root, acc1[0], acc1[1])

    h2 = _relmm(nrel, npad, dh, dwout)(h, w2_p)
    acc2 = _aggregate(npad, erows_pad, dwout)(
        h2.reshape(nrel * npad, dwout), fsrc_p, dst_p, norm_p)
    z = _root_fuse(npad, dh, dwout, False)(h, w2r_p, acc2[0], acc2[1])

    idx2d = jnp.concatenate(
        [indice_pairs[:, 0], indice_pairs[:, 1]]).reshape(-1, 128)
    g = _pair_gather(idx2d.shape[0], dwout)(z, idx2d)
    z1 = g[:npairs, :dout]
    z2 = g[npairs:, :dout]
    return (z1, z2)


# speedup vs baseline: 33.6539x; 1.2060x over previous
import functools

import jax
import jax.numpy as jnp
from jax import lax
from jax.experimental import pallas as pl
from jax.experimental.pallas import tpu as pltpu
from jax.experimental.pallas import tpu_sc as plsc

_NC = 2
_NS = 16
_NW = _NC * _NS
_L = 16


def _mesh():
    return plsc.VectorSubcoreMesh(core_axis_name="c", subcore_axis_name="s")


def _edge_prep(npad, erows, deg_size):
    out_type = (
        jax.ShapeDtypeStruct((erows, 128), jnp.int32),
        jax.ShapeDtypeStruct((erows, 128), jnp.int32),
        jax.ShapeDtypeStruct((erows, 128), jnp.float32),
    )
    deg_rows = deg_size // 128
    nslices = 20
    assert deg_rows % nslices == 0
    srows = deg_rows // nslices
    assert srows % _L == 0 and deg_rows % _NS == 0

    @functools.partial(
        pl.kernel,
        out_type=out_type,
        mesh=_mesh(),
        compiler_params=pltpu.CompilerParams(needs_layout_passes=False),
        scratch_types=[
            pltpu.VMEM_SHARED((deg_rows, 128), jnp.float32),
            pltpu.VMEM((deg_rows, 128), jnp.float32),
            pltpu.VMEM((srows,), jnp.int32),
            pltpu.VMEM((128,), jnp.int32),
            pltpu.VMEM((128,), jnp.int32),
            pltpu.VMEM((128,), jnp.int32),
            pltpu.VMEM((128,), jnp.int32),
            pltpu.VMEM((128,), jnp.float32),
        ],
    )
    def prep(type2d, dst2d, src2d, fsrc_out, dst_out, norm_out,
             deg_sh, deg_l, ridx_v, t_v, d_v, s_v, fsrc_v, norm_v):
        c = lax.axis_index("c")
        s = lax.axis_index("s")
        wid = s * _NC + c

        def zz(i, carry):
            for j in range(128 // _L):
                deg_l[i, pl.ds(j * _L, _L)] = jnp.zeros((_L,), jnp.float32)
            return carry
        lax.fori_loop(0, deg_rows, zz, None)

        zrows = deg_rows // _NS
        pltpu.sync_copy(deg_l.at[pl.ds(s * zrows, zrows)],
                        deg_sh.at[pl.ds(s * zrows, zrows)])
        plsc.subcore_barrier()

        ones = jnp.ones((_L,), jnp.float32)
        nrows_deg = (erows - s + _NS - 1) // _NS

        def deg_body(i, carry):
            row = s + i * _NS
            pltpu.sync_copy(type2d.at[row], t_v)
            pltpu.sync_copy(dst2d.at[row], d_v)
            for j in range(128 // _L):
                sl = pl.ds(j * _L, _L)
                fdst = t_v[sl] * npad + d_v[sl]
                plsc.addupdate_scatter(
                    deg_l, [lax.shift_right_logical(fdst, 7), fdst & 127],
                    ones)
            return carry
        lax.fori_loop(0, nrows_deg, deg_body, None)

        iota16 = lax.iota(jnp.int32, _L)
        for k in range(nslices):
            b = lax.rem(s + k, nslices)
            for j in range(srows // _L):
                ridx_v[pl.ds(j * _L, _L)] = b * srows + j * _L + iota16
            pltpu.sync_copy(deg_l.at[pl.ds(b * srows, srows)],
                            deg_sh.at[ridx_v], add=True)
        plsc.subcore_barrier()

        pltpu.sync_copy(deg_sh, deg_l)

        nrows_w = (erows - wid + _NW - 1) // _NW

        def norm_body(i, carry):
            row = wid + i * _NW
            pltpu.sync_copy(type2d.at[row], t_v)
            pltpu.sync_copy(dst2d.at[row], d_v)
            pltpu.sync_copy(src2d.at[row], s_v)
            for j in range(128 // _L):
                sl = pl.ds(j * _L, _L)
                t = t_v[sl]
                fdst = t * npad + d_v[sl]
                fsrc_v[sl] = t * npad + s_v[sl]
                deg = plsc.load_gather(
                    deg_l, [lax.shift_right_logical(fdst, 7), fdst & 127])
                norm_v[sl] = 1.0 / jnp.maximum(deg, 1.0)
            pltpu.sync_copy(fsrc_v, fsrc_out.at[row])
            pltpu.sync_copy(d_v, dst_out.at[row])
            pltpu.sync_copy(norm_v, norm_out.at[row])
            return carry
        lax.fori_loop(0, nrows_w, norm_body, None)

    return prep


def _aggregate(npad, erows_pad, dw):
    rows_per_tile = erows_pad // _NW
    assert rows_per_tile % 2 == 0
    npairs = rows_per_tile // 2
    acc_rows = npad // _NS
    nch = dw // _L

    @functools.partial(
        pl.kernel,
        out_type=jax.ShapeDtypeStruct((_NC, npad, dw), jnp.float32),
        mesh=_mesh(),
        scratch_types=[
            pltpu.VMEM_SHARED((npad, dw), jnp.float32),
            pltpu.VMEM((128, dw), jnp.float32),
            pltpu.VMEM((128, dw), jnp.float32),
            pltpu.VMEM((128,), jnp.int32),
            pltpu.VMEM((128,), jnp.int32),
            pltpu.VMEM((128,), jnp.int32),
            pltpu.VMEM((128,), jnp.int32),
            pltpu.VMEM((128,), jnp.float32),
            pltpu.VMEM((128,), jnp.float32),
            pltpu.SemaphoreType.DMA,
            pltpu.SemaphoreType.DMA,
        ],
    )
    def agg(h_flat, fsrc_p, dst_p, norm_p, out, acc_sh,
            rows_a, rows_b, fsrc_a, fsrc_b, dst_a, dst_b,
            norm_a, norm_b, sem_a, sem_b):
        c = lax.axis_index("c")
        s = lax.axis_index("s")
        wid = s * _NC + c
        base = wid * rows_per_tile

        def zrow(k, carry):
            for j in range(nch):
                rows_a[k, pl.ds(j * _L, _L)] = jnp.zeros((_L,), jnp.float32)
            return carry
        lax.fori_loop(0, 128, zrow, None)

        def zacc(i, carry):
            pltpu.sync_copy(rows_a,
                            acc_sh.at[pl.ds(s * acc_rows + i * 128, 128)])
            return carry
        lax.fori_loop(0, acc_rows // 128, zacc, None)
        plsc.subcore_barrier()

        def load_idx(row, fsrc_v, dst_v, norm_v):
            pltpu.sync_copy(fsrc_p.at[row], fsrc_v)
            pltpu.sync_copy(dst_p.at[row], dst_v)
            pltpu.sync_copy(norm_p.at[row], norm_v)

        def scale(rows_v, norm_v):
            def sbody(g, carry):
                nv16 = norm_v[pl.ds(g * _L, _L)]
                for k16 in range(_L):
                    k = g * _L + k16
                    nvs = jnp.full((_L,), nv16[k16], jnp.float32)
                    for j in range(nch):
                        sl = pl.ds(j * _L, _L)
                        rows_v[k, sl] = rows_v[k, sl] * nvs
                return carry
            lax.fori_loop(0, 128 // _L, sbody, None)

        load_idx(base, fsrc_a, dst_a, norm_a)
        ga0 = pltpu.async_copy(h_flat.at[fsrc_a], rows_a, sem_a)

        def body(i, carry):
            row_e = base + 2 * i
            load_idx(row_e + 1, fsrc_b, dst_b, norm_b)
            pltpu.async_copy(h_flat.at[fsrc_b], rows_b, sem_b)
            pltpu.make_async_copy(h_flat.at[fsrc_a], rows_a, sem_a).wait()
            scale(rows_a, norm_a)
            pltpu.sync_copy(rows_a, acc_sh.at[dst_a], add=True)
            @pl.when(i + 1 < npairs)
            def _():
                load_idx(row_e + 2, fsrc_a, dst_a, norm_a)
                pltpu.async_copy(h_flat.at[fsrc_a], rows_a, sem_a)
            pltpu.make_async_copy(h_flat.at[fsrc_b], rows_b, sem_b).wait()
            scale(rows_b, norm_b)
            pltpu.sync_copy(rows_b, acc_sh.at[dst_b], add=True)
            return carry
        lax.fori_loop(0, npairs, body, None)
        plsc.subcore_barrier()

        def drain(i, carry):
            sl = pl.ds(s * acc_rows + i * 128, 128)
            pltpu.sync_copy(acc_sh.at[sl], out.at[c, sl])
            return carry
        lax.fori_loop(0, acc_rows // 128, drain, None)

    return agg


def _pair_gather(nidx_rows, dw):
    rpt = nidx_rows // _NW

    @functools.partial(
        pl.kernel,
        out_type=jax.ShapeDtypeStruct((nidx_rows * 128, dw), jnp.float32),
        mesh=_mesh(),
        scratch_types=[
            pltpu.VMEM((128,), jnp.int32),
            pltpu.VMEM((128, dw), jnp.float32),
            pltpu.SemaphoreType.DMA,
        ],
    )
    def gk(z_hbm, idx2d, out, idx_v, rows_v, sem):
        c = lax.axis_index("c")
        s = lax.axis_index("s")
        wid = s * _NC + c

        def body(i, carry):
            row = wid * rpt + i
            pltpu.sync_copy(idx2d.at[row], idx_v)
            pltpu.async_copy(z_hbm.at[idx_v], rows_v, sem).wait()
            pltpu.sync_copy(rows_v, out.at[pl.ds(row * 128, 128)])
            return carry
        lax.fori_loop(0, rpt, body, None)

    return gk


def _relmm(nrel, npad, din, dh):
    def mmk(x_ref, w_ref, o_ref):
        for r in range(nrel):
            o_ref[r] = jnp.dot(x_ref[...], w_ref[r],
                               preferred_element_type=jnp.float32)

    return pl.pallas_call(
        mmk,
        grid=(npad // 128,),
        in_specs=[
            pl.BlockSpec((128, din), lambda n: (n, 0)),
            pl.BlockSpec((nrel, din, dh), lambda n: (0, 0, 0)),
        ],
        out_specs=pl.BlockSpec((nrel, 128, dh), lambda n: (0, n, 0)),
        out_shape=jax.ShapeDtypeStruct((nrel, npad, dh), jnp.float32),
    )


def _root_fuse(npad, din, dh, relu):
    def k(x_ref, w_ref, a_ref, b_ref, o_ref):
        acc = jnp.dot(x_ref[...], w_ref[...],
                      preferred_element_type=jnp.float32)
        acc = acc + a_ref[...] + b_ref[...]
        o_ref[...] = jnp.maximum(acc, 0.0) if relu else acc

    return pl.pallas_call(
        k,
        grid=(npad // 128,),
        in_specs=[
            pl.BlockSpec((128, din), lambda n: (n, 0)),
            pl.BlockSpec((din, dh), lambda n: (0, 0)),
            pl.BlockSpec((128, dh), lambda n: (n, 0)),
            pl.BlockSpec((128, dh), lambda n: (n, 0)),
        ],
        out_specs=pl.BlockSpec((128, dh), lambda n: (n, 0)),
        out_shape=jax.ShapeDtypeStruct((npad, dh), jnp.float32),
    )


def kernel(x, edge_index, edge_type, indice_pairs, W1, W1_root, W2, W2_root):
    n_nodes, din = x.shape
    n_edges = edge_type.shape[0]
    nrel = W1.shape[0]
    dh = W1.shape[2]
    dout = W2.shape[2]
    npairs = indice_pairs.shape[0]

    assert n_edges % 128 == 0 and (2 * npairs) % (128 * _NW) == 0
    npad = -(-n_nodes // 2048) * 2048
    erows = n_edges // 128
    erows_pad = -(-erows // 64) * 64
    dwout = -(-dout // 128) * 128
    deg_size = nrel * npad

    x_p = jnp.pad(x, ((0, npad - n_nodes), (0, 0)))
    type2d = edge_type.reshape(erows, 128)
    src2d = edge_index[0].reshape(erows, 128)
    dst2d = edge_index[1].reshape(erows, 128)
    w2_p = jnp.pad(W2, ((0, 0), (0, 0), (0, dwout - dout)))
    w2r_p = jnp.pad(W2_root, ((0, 0), (0, dwout - dout)))

    fsrc_p, dst_p, norm_p = _edge_prep(npad, erows, deg_size)(
        type2d, dst2d, src2d)

    pad_rows = erows_pad - erows
    if pad_rows:
        iota_row = jnp.tile(jnp.arange(128, dtype=jnp.int32), (pad_rows, 1))
        fsrc_p = jnp.concatenate([fsrc_p, iota_row], axis=0)
        dst_p = jnp.concatenate([dst_p, iota_row], axis=0)
        norm_p = jnp.concatenate(
            [norm_p, jnp.zeros((pad_rows, 128), jnp.float32)], axis=0)

    h1 = _relmm(nrel, npad, din, dh)(x_p, W1)
    acc1 = _aggregate(npad, erows_pad, dh)(
        h1.reshape(nrel * npad, dh), fsrc_p, dst_p, norm_p)
    h = _root_fuse(npad, din, dh, True)(x_p, W1_root, acc1[0], acc1[1])

    h2 = _relmm(nrel, npad, dh, dwout)(h, w2_p)
    acc2 = _aggregate(npad, erows_pad, dwout)(
        h2.reshape(nrel * npad, dwout), fsrc_p, dst_p, norm_p)
    z = _root_fuse(npad, dh, dwout, False)(h, w2r_p, acc2[0], acc2[1])

    idx2d = jnp.concatenate(
        [indice_pairs[:, 0], indice_pairs[:, 1]]).reshape(-1, 128)
    g = _pair_gather(idx2d.shape[0], dwout)(z, idx2d)
    z1 = g[:npairs, :dout]
    z2 = g[npairs:, :dout]
    return (z1, z2)
